# baseline (speedup 1.0000x reference)
import functools

import jax
import jax.numpy as jnp
from jax import lax
from jax.experimental import pallas as pl
from jax.experimental.pallas import tpu as pltpu
from jax.experimental.pallas import tpu_sc as plsc

L = 16
TW = 128


@functools.lru_cache(maxsize=None)
def _make(M: int, D: int, B: int):
  info = plsc.get_sparse_core_info()
  NW = info.num_cores * info.num_subcores
  NTC = (M + TW - 1) // TW
  RAG = M % TW
  NTC_FULL = NTC - (1 if RAG else 0)
  MAXT = -(-NTC // NW)
  TAGN = MAXT * TW
  NV = B // L
  assert B % L == 0 and D % L == 0 and B % 2 == 0
  mesh = plsc.VectorSubcoreMesh(core_axis_name="c", subcore_axis_name="s")

  @functools.partial(
      pl.kernel,
      out_type=jax.ShapeDtypeStruct((D, M), jnp.float32),
      mesh=mesh,
      compiler_params=pltpu.CompilerParams(needs_layout_passes=False),
      scratch_types=[
          pltpu.VMEM((B,), jnp.int32),
          pltpu.VMEM((TAGN,), jnp.int32),
          pltpu.VMEM((TW,), jnp.int32),
          pltpu.VMEM((TW,), jnp.int32),
          pltpu.VMEM((TW,), jnp.int32),
          pltpu.VMEM((TW, TW), jnp.float32),
          pltpu.VMEM((D, TW), jnp.float32),
          pltpu.VMEM((D, TW), jnp.float32),
          pltpu.VMEM((D, RAG or 1), jnp.float32),
          pltpu.SemaphoreType.DMA,
          pltpu.SemaphoreType.DMA,
          pltpu.SemaphoreType.DMA,
          pltpu.SemaphoreType.DMA,
          pltpu.SemaphoreType.DMA,
      ],
  )
  def sweep(memT, idxr, val2, outT, idx_v, tag_v, gidx_v, half_v, cols_v,
            vrows_v, tbuf0, tbuf1, rbuf, insem0, insem1, outsem0, outsem1,
            gsem):
    wid = lax.axis_index("s") * info.num_cores + lax.axis_index("c")
    tc0 = (wid * NTC) // NW
    tc1 = ((wid + 1) * NTC) // NW
    iota = lax.iota(jnp.int32, L)
    neg1 = jnp.full((L,), -1, jnp.int32)
    ones = jnp.full((L,), 1, jnp.int32)
    zeros = jnp.zeros((L,), jnp.int32)
    basec = tc0 * TW
    hic = tc1 * TW

    pltpu.sync_copy(idxr, idx_v)

    def init_tags(i, _):
      tag_v[pl.ds(i * L, L)] = zeros
      return ()
    lax.fori_loop(0, TAGN // L, init_tags, ())

    def scan(i, _):
      idxv = idx_v[pl.ds(i * L, L)]
      owned = (idxv >= basec) & (idxv < hic)
      _, lastm = plsc.scan_count(idxv, mask=owned)
      m = lastm & owned
      plsc.store_scatter(tag_v, [idxv - basec], iota + (i * L + 1), mask=m)
      return ()
    lax.fori_loop(0, NV, scan, ())

    def compact(j):
      off = zeros
      for k in range(TW // L):
        gidx_v[pl.ds(k * L, L)] = neg1
      for k in range(TW // L):
        tags = tag_v[pl.ds(j * TW + k * L, L)]
        m = tags > 0
        rank = plsc.cumsum(ones, mask=m)
        dst = off + rank - 1
        p = tags - 1
        plsc.store_scatter(gidx_v, [dst], lax.shift_right_logical(p, 1),
                           mask=m)
        plsc.store_scatter(half_v, [dst], p & 1, mask=m)
        plsc.store_scatter(cols_v, [dst], iota + k * L, mask=m)
        off = off + plsc.all_reduce_population_count(m)
      return off[0]

    def fetch_rows(cnt):
      @pl.when(cnt > 0)
      def _():
        pltpu.async_copy(
            val2.at[plsc.Indices(gidx_v, ignored_value=-1)], vrows_v,
            gsem).wait()

    def patch(buf, cnt):
      ng = (cnt + (L - 1)) // L
      def group(g, _):
        lanes = iota + g * L
        valid = lanes < cnt
        colg = cols_v[pl.ds(g * L, L)]
        srcc = half_v[pl.ds(g * L, L)] * D
        for d in range(D):
          sv = plsc.load_gather(vrows_v, [lanes, srcc + d], mask=valid)
          plsc.store_scatter(buf, [jnp.full((L,), d, jnp.int32), colg], sv,
                             mask=valid)
        return ()
      lax.fori_loop(0, ng, group, ())

    nt = lax.min(tc1, NTC_FULL) - tc0

    @pl.when(nt > 0)
    def _():
      pltpu.async_copy(memT.at[:, pl.ds(tc0 * TW, TW)], tbuf0, insem0)

    def step(j, bufs):
      buf, insem, outsem, obuf, oinsem, ooutsem = bufs
      tcg = tc0 + j

      @pl.when(j >= 1)
      def _():
        pltpu.make_async_copy(obuf, outT.at[:, pl.ds(0, TW)], ooutsem).wait()

      @pl.when(j + 1 < nt)
      def _():
        pltpu.async_copy(memT.at[:, pl.ds((tcg + 1) * TW, TW)], obuf, oinsem)

      cnt = compact(j)
      fetch_rows(cnt)
      pltpu.make_async_copy(memT.at[:, pl.ds(0, TW)], buf, insem).wait()
      patch(buf, cnt)
      pltpu.async_copy(buf, outT.at[:, pl.ds(tcg * TW, TW)], outsem)

    def loop(j, _):
      @pl.when(j % 2 == 0)
      def _():
        step(j, (tbuf0, insem0, outsem0, tbuf1, insem1, outsem1))
      @pl.when(j % 2 == 1)
      def _():
        step(j, (tbuf1, insem1, outsem1, tbuf0, insem0, outsem0))
      return ()
    lax.fori_loop(0, nt, loop, ())

    @pl.when(nt > 0)
    def _():
      @pl.when(nt % 2 == 1)
      def _():
        pltpu.make_async_copy(tbuf0, outT.at[:, pl.ds(0, TW)], outsem0).wait()
      @pl.when(nt % 2 == 0)
      def _():
        pltpu.make_async_copy(tbuf1, outT.at[:, pl.ds(0, TW)], outsem1).wait()

    if RAG:
      @pl.when(tc1 * TW > M)
      def _():
        jr = NTC_FULL - tc0
        pltpu.sync_copy(memT.at[:, pl.ds(NTC_FULL * TW, RAG)], rbuf)
        cnt = compact(jr)
        fetch_rows(cnt)
        patch(rbuf, cnt)
        pltpu.sync_copy(rbuf, outT.at[:, pl.ds(NTC_FULL * TW, RAG)])

  return sweep


def kernel(mem, idx, val):
  M, D = mem.shape
  (B,) = idx.shape
  memT = mem.T
  val2 = val.reshape(B // 2, 2 * D)
  idx32 = idx.astype(jnp.int32)
  outT = _make(M, D, B)(memT, idx32, val2)
  return outT.T


# speedup vs baseline: 1.3591x; 1.3591x over previous
import functools

import jax
import jax.numpy as jnp
from jax import lax
from jax.experimental import pallas as pl
from jax.experimental.pallas import tpu as pltpu
from jax.experimental.pallas import tpu_sc as plsc

L = 16
TW = 128


@functools.lru_cache(maxsize=None)
def _make(M: int, D: int, B: int):
  info = plsc.get_sparse_core_info()
  NW = info.num_cores * info.num_subcores
  NTC = (M + TW - 1) // TW
  RAG = M % TW
  NTC_FULL = NTC - (1 if RAG else 0)
  MAXT = -(-NTC // NW)
  TAGN = MAXT * TW
  NV = B // L
  assert B % L == 0 and D % L == 0 and B % 2 == 0 and TW % L == 0
  mesh = plsc.VectorSubcoreMesh(core_axis_name="c", subcore_axis_name="s")

  @functools.partial(
      pl.kernel,
      out_type=jax.ShapeDtypeStruct((D, M), jnp.float32),
      mesh=mesh,
      compiler_params=pltpu.CompilerParams(needs_layout_passes=False),
      scratch_types=[
          pltpu.VMEM((B,), jnp.int32),
          pltpu.VMEM((TAGN,), jnp.int32),
          pltpu.VMEM((B,), jnp.int32),
          pltpu.VMEM((256,), jnp.int32),
          pltpu.VMEM((TW,), jnp.int32),
          pltpu.VMEM((TW,), jnp.int32),
          pltpu.VMEM((TW, TW), jnp.float32),
          pltpu.VMEM((TW, TW), jnp.float32),
          pltpu.VMEM((D, TW), jnp.float32),
          pltpu.VMEM((D, TW), jnp.float32),
          pltpu.VMEM((D, RAG or 1), jnp.float32),
          pltpu.SemaphoreType.DMA,
          pltpu.SemaphoreType.DMA,
          pltpu.SemaphoreType.DMA,
          pltpu.SemaphoreType.DMA,
          pltpu.SemaphoreType.DMA,
          pltpu.SemaphoreType.DMA,
      ],
  )
  def sweep(memT, idxr, val2, outT, idx_v, tag_v, wlist_v, off_v,
            gidxA, gidxB, vrowsA, vrowsB, tbuf0, tbuf1, rbuf,
            insem0, insem1, outsem0, outsem1, gsemA, gsemB):
    wid = lax.axis_index("s") * info.num_cores + lax.axis_index("c")
    tc0 = (wid * NTC) // NW
    tc1 = ((wid + 1) * NTC) // NW
    iota = lax.iota(jnp.int32, L)
    neg1 = jnp.full((L,), -1, jnp.int32)
    ones = jnp.full((L,), 1, jnp.int32)
    zeros = jnp.zeros((L,), jnp.int32)
    lane0 = iota == 0
    basec = tc0 * TW
    hic = tc1 * TW
    ntw = tc1 - tc0
    nt_main = jnp.minimum(tc1, NTC_FULL) - tc0

    @pl.when(nt_main > 0)
    def _():
      pltpu.async_copy(memT.at[:, pl.ds(tc0 * TW, TW)], tbuf0, insem0)

    pltpu.sync_copy(idxr, idx_v)

    def init_tags(i, _):
      tag_v[pl.ds(i * L, L)] = zeros
      return ()
    lax.fori_loop(0, TAGN // L, init_tags, ())

    def scan(i, _):
      idxv = idx_v[pl.ds(i * L, L)]
      owned = (idxv >= basec) & (idxv < hic)
      _, lastm = plsc.scan_count(idxv, mask=owned)
      m = lastm & owned
      plsc.store_scatter(tag_v, [idxv - basec], iota + (i * L + 1), mask=m)
      return ()
    lax.fori_loop(0, NV, scan, ())

    def csr_outer(j, off):
      plsc.store_scatter(off_v, [jnp.full((L,), j, jnp.int32)], off,
                         mask=lane0)
      for k in range(TW // L):
        tags = tag_v[pl.ds(j * TW + k * L, L)]
        m = tags > 0
        pc = plsc.all_reduce_population_count(m)
        off_k = off
        @pl.when(pc[0] > 0)
        def _():
          rank = plsc.cumsum(ones, mask=m)
          dst = off_k + rank - 1
          packed = ((tags - 1) << 7) | (iota + k * L)
          plsc.store_scatter(wlist_v, [dst], packed, mask=m)
        off = off + pc
      return off
    off_fin = lax.fori_loop(0, ntw, csr_outer, zeros)
    plsc.store_scatter(off_v, [jnp.full((L,), ntw, jnp.int32)], off_fin,
                       mask=lane0)

    def offs_at(j):
      return plsc.load_gather(off_v, [jnp.full((L,), j, jnp.int32)])

    def fire_gather(c0v, cnt, gidx_b, vrows_b, gsem_b):
      for k in range(TW // L):
        gidx_b[pl.ds(k * L, L)] = neg1
      ng = (cnt + (L - 1)) // L
      def g(gi, _):
        lanes = iota + gi * L
        valid = lanes < cnt
        packed = plsc.load_gather(wlist_v, [c0v + lanes], mask=valid)
        plsc.store_scatter(gidx_b, [lanes],
                           lax.shift_right_logical(packed, 8), mask=valid)
        return ()
      lax.fori_loop(0, ng, g, ())
      pltpu.async_copy(val2.at[plsc.Indices(gidx_b, ignored_value=-1)],
                       vrows_b, gsem_b)

    def wait_gather(gidx_b, vrows_b, gsem_b):
      pltpu.make_async_copy(
          val2.at[plsc.Indices(gidx_b, ignored_value=-1)], vrows_b,
          gsem_b).wait()

    def patch(buf, c0v, cnt, vrows_b):
      ng = (cnt + (L - 1)) // L
      def g(gi, _):
        lanes = iota + gi * L
        valid = lanes < cnt
        packed = plsc.load_gather(wlist_v, [c0v + lanes], mask=valid)
        colg = packed & (TW - 1)
        srcc = (lax.shift_right_logical(packed, 7) & 1) * D
        for d in range(D):
          sv = plsc.load_gather(vrows_b, [lanes, srcc + d], mask=valid)
          plsc.store_scatter(buf, [jnp.full((L,), d, jnp.int32), colg], sv,
                             mask=valid)
        return ()
      lax.fori_loop(0, ng, g, ())

    c0v0 = offs_at(0)
    cntv0 = offs_at(1) - c0v0
    @pl.when((nt_main > 0) & (cntv0[0] > 0))
    def _():
      fire_gather(c0v0, cntv0[0], gidxA, vrowsA, gsemA)

    def loop(j, carry):
      c0v, cntv = carry
      n0 = offs_at(j + 1)
      n1 = offs_at(j + 2)
      cnt = cntv[0]
      cntn = (n1 - n0)[0]

      def halfstep(buf, insem, outsem, obuf, oinsem, ooutsem,
                   gidx_b, vrows_b, gsem_b, ogidx, ovrows, ogsem):
        tcg = tc0 + j

        @pl.when(j >= 1)
        def _():
          pltpu.make_async_copy(obuf, outT.at[:, pl.ds(0, TW)],
                                ooutsem).wait()

        @pl.when(j + 1 < nt_main)
        def _():
          pltpu.async_copy(memT.at[:, pl.ds((tcg + 1) * TW, TW)], obuf,
                           oinsem)

        @pl.when((j + 1 < nt_main) & (cntn > 0))
        def _():
          fire_gather(n0, cntn, ogidx, ovrows, ogsem)

        pltpu.make_async_copy(memT.at[:, pl.ds(0, TW)], buf, insem).wait()

        @pl.when(cnt > 0)
        def _():
          wait_gather(gidx_b, vrows_b, gsem_b)
          patch(buf, c0v, cnt, vrows_b)

        pltpu.async_copy(buf, outT.at[:, pl.ds(tcg * TW, TW)], outsem)

      @pl.when(j % 2 == 0)
      def _():
        halfstep(tbuf0, insem0, outsem0, tbuf1, insem1, outsem1,
                 gidxA, vrowsA, gsemA, gidxB, vrowsB, gsemB)
      @pl.when(j % 2 == 1)
      def _():
        halfstep(tbuf1, insem1, outsem1, tbuf0, insem0, outsem0,
                 gidxB, vrowsB, gsemB, gidxA, vrowsA, gsemA)
      return (n0, n1 - n0)
    lax.fori_loop(0, nt_main, loop, (c0v0, cntv0))

    @pl.when(nt_main > 0)
    def _():
      @pl.when(nt_main % 2 == 1)
      def _():
        pltpu.make_async_copy(tbuf0, outT.at[:, pl.ds(0, TW)],
                              outsem0).wait()
      @pl.when(nt_main % 2 == 0)
      def _():
        pltpu.make_async_copy(tbuf1, outT.at[:, pl.ds(0, TW)],
                              outsem1).wait()

    if RAG:
      @pl.when(tc1 * TW > M)
      def _():
        pltpu.sync_copy(memT.at[:, pl.ds(NTC_FULL * TW, RAG)], rbuf)
        c0r = offs_at(ntw - 1)
        cntr = (offs_at(ntw) - c0r)[0]
        @pl.when(cntr > 0)
        def _():
          fire_gather(c0r, cntr, gidxA, vrowsA, gsemA)
          wait_gather(gidxA, vrowsA, gsemA)
          patch(rbuf, c0r, cntr, vrowsA)
        pltpu.sync_copy(rbuf, outT.at[:, pl.ds(NTC_FULL * TW, RAG)])

  return sweep


def kernel(mem, idx, val):
  M, D = mem.shape
  (B,) = idx.shape
  memT = mem.T
  val2 = val.reshape(B // 2, 2 * D)
  idx32 = idx.astype(jnp.int32)
  outT = _make(M, D, B)(memT, idx32, val2)
  return outT.T
